# baseline (speedup 1.0000x reference)
import functools

import jax
import jax.numpy as jnp
from jax import lax
from jax.experimental import pallas as pl
from jax.experimental.pallas import tpu as pltpu
from jax.experimental.pallas import tpu_sc as plsc

VOCAB = 100000
EMB_DIM = 64
BATCH = 1024
N_BLK = 2048


@functools.lru_cache(maxsize=None)
def _make_gather():
    info = plsc.get_sparse_core_info()
    nc, ns = info.num_cores, info.num_subcores
    nw = nc * ns
    b_per_w = BATCH // nw
    p_per_w = b_per_w // 2
    mesh = plsc.VectorSubcoreMesh(core_axis_name="c", subcore_axis_name="s")

    @functools.partial(
        pl.kernel,
        mesh=mesh,
        out_type=jax.ShapeDtypeStruct((BATCH // 2, 2 * EMB_DIM), jnp.float32),
        scratch_types=[
            pltpu.VMEM((b_per_w,), jnp.int32),
            pltpu.VMEM((b_per_w,), jnp.int32),
            pltpu.VMEM((b_per_w, 2 * EMB_DIM), jnp.float32),
            pltpu.VMEM((p_per_w, 2 * EMB_DIM), jnp.float32),
            pltpu.SemaphoreType.DMA,
        ],
        compiler_params=pltpu.CompilerParams(
            use_tc_tiling_on_sc=True, needs_layout_passes=False),
    )
    def gather_kernel(tbl_hbm, idx_hbm, out_hbm, idx_v, pair_v, rows_v,
                      sel_v, sem):
        wid = lax.axis_index("s") * nc + lax.axis_index("c")
        base = wid * b_per_w
        pltpu.sync_copy(idx_hbm.at[pl.ds(base, b_per_w)], idx_v)
        for j0 in range(0, b_per_w, 16):
            pair_v[pl.ds(j0, 16)] = lax.shift_right_logical(
                idx_v[pl.ds(j0, 16)], 1)
        pltpu.async_copy(tbl_hbm.at[pair_v], rows_v, sem).wait()
        iota = lax.iota(jnp.int32, 16)
        for j0 in range(0, b_per_w, 16):
            jv = iota + j0
            half = (idx_v[pl.ds(j0, 16)] & 1) * EMB_DIM
            dst_row = lax.shift_right_logical(jv, 1)
            dst_col0 = (jv & 1) * EMB_DIM
            for c in range(EMB_DIM):
                val = plsc.load_gather(rows_v, [jv, half + c])
                plsc.store_scatter(sel_v, [dst_row, dst_col0 + c], val)
        pltpu.sync_copy(sel_v, out_hbm.at[pl.ds(wid * p_per_w, p_per_w)])

    return gather_kernel


def _gather(indices, embedding_weight):
    tbl = jnp.concatenate(
        [embedding_weight[0::2], embedding_weight[1::2]], axis=1)
    packed = _make_gather()(tbl, indices)
    return packed.reshape(BATCH, EMB_DIM)


def _matmul_block(wt_ref, emb_ref, out_ref):
    wt = wt_ref[...].astype(jnp.bfloat16)
    e = emb_ref[...].astype(jnp.bfloat16)
    out_ref[...] = lax.dot_general(
        wt, e, (((0,), (1,)), ((), ())), preferred_element_type=jnp.float32)


def _project(emb, emb2vocab_weight, interpret=False):
    out_t = pl.pallas_call(
        _matmul_block,
        grid=(pl.cdiv(VOCAB, N_BLK),),
        in_specs=[
            pl.BlockSpec((EMB_DIM, N_BLK), lambda i: (0, i)),
            pl.BlockSpec((BATCH, EMB_DIM), lambda i: (0, 0)),
        ],
        out_specs=pl.BlockSpec((N_BLK, BATCH), lambda i: (i, 0)),
        out_shape=jax.ShapeDtypeStruct((VOCAB, BATCH), jnp.float32),
        compiler_params=pltpu.CompilerParams(
            vmem_limit_bytes=100 * 1024 * 1024,
        ),
        interpret=interpret,
    )(emb2vocab_weight.T, emb)
    return out_t.T


def kernel(indices, embedding_weight, emb2vocab_weight):
    emb = _gather(indices, embedding_weight)
    return _project(emb, emb2vocab_weight)


# speedup vs baseline: 4.9132x; 4.9132x over previous
import functools

import jax
import jax.numpy as jnp
from jax import lax
from jax.experimental import pallas as pl
from jax.experimental.pallas import tpu as pltpu
from jax.experimental.pallas import tpu_sc as plsc

VOCAB = 100000
EMB_DIM = 64
BATCH = 1024
N_BLK = 2048


@functools.lru_cache(maxsize=None)
def _make_gather():
    info = plsc.get_sparse_core_info()
    nc, ns = info.num_cores, info.num_subcores
    nw = nc * ns
    b_per_w = BATCH // nw
    p_per_w = b_per_w // 2
    mesh = plsc.VectorSubcoreMesh(core_axis_name="c", subcore_axis_name="s")

    @functools.partial(
        pl.kernel,
        mesh=mesh,
        out_type=jax.ShapeDtypeStruct((BATCH // 2, 2 * EMB_DIM), jnp.float32),
        scratch_types=[
            pltpu.VMEM((b_per_w,), jnp.int32),
            pltpu.VMEM((b_per_w, 2 * EMB_DIM), jnp.float32),
            pltpu.VMEM((p_per_w, 2 * EMB_DIM), jnp.float32),
            pltpu.SemaphoreType.DMA,
        ],
        compiler_params=pltpu.CompilerParams(
            use_tc_tiling_on_sc=True, needs_layout_passes=False),
    )
    def gather_kernel(tbl_hbm, idx_hbm, out_hbm, idx_v, rows_v,
                      sel_v, sem):
        wid = lax.axis_index("s") * nc + lax.axis_index("c")
        base = wid * b_per_w
        pltpu.sync_copy(idx_hbm.at[pl.ds(base, b_per_w)], idx_v)
        pltpu.async_copy(tbl_hbm.at[idx_v], rows_v, sem).wait()
        iota = lax.iota(jnp.int32, 16)
        for j0 in range(0, b_per_w, 16):
            jv = iota + j0
            dst_row = lax.shift_right_logical(jv, 1)
            dst_col0 = (jv & 1) * EMB_DIM
            for c in range(EMB_DIM):
                val = plsc.load_gather(rows_v, [jv, iota * 0 + c])
                plsc.store_scatter(sel_v, [dst_row, dst_col0 + c], val)
        pltpu.sync_copy(sel_v, out_hbm.at[pl.ds(wid * p_per_w, p_per_w)])

    return gather_kernel


def _gather(indices, embedding_weight):
    tbl = jnp.pad(embedding_weight, ((0, 0), (0, 2 * EMB_DIM - EMB_DIM)))
    packed = _make_gather()(tbl, indices)
    return packed.reshape(BATCH, EMB_DIM)


def _matmul_block(wt_ref, emb_ref, out_ref):
    wt = wt_ref[...].astype(jnp.bfloat16)
    e = emb_ref[...].astype(jnp.bfloat16)
    out_ref[...] = lax.dot_general(
        wt, e, (((0,), (1,)), ((), ())), preferred_element_type=jnp.float32)


def _project(emb, emb2vocab_weight, interpret=False):
    out_t = pl.pallas_call(
        _matmul_block,
        grid=(pl.cdiv(VOCAB, N_BLK),),
        in_specs=[
            pl.BlockSpec((EMB_DIM, N_BLK), lambda i: (0, i)),
            pl.BlockSpec((BATCH, EMB_DIM), lambda i: (0, 0)),
        ],
        out_specs=pl.BlockSpec((N_BLK, BATCH), lambda i: (i, 0)),
        out_shape=jax.ShapeDtypeStruct((VOCAB, BATCH), jnp.float32),
        compiler_params=pltpu.CompilerParams(
            vmem_limit_bytes=100 * 1024 * 1024,
        ),
        interpret=interpret,
    )(emb2vocab_weight.T, emb)
    return out_t.T


def kernel(indices, embedding_weight, emb2vocab_weight):
    emb = _gather(indices, embedding_weight)
    return _project(emb, emb2vocab_weight)


# speedup vs baseline: 4.9195x; 1.0013x over previous
import functools

import jax
import jax.numpy as jnp
from jax import lax
from jax.experimental import pallas as pl
from jax.experimental.pallas import tpu as pltpu
from jax.experimental.pallas import tpu_sc as plsc

VOCAB = 100000
EMB_DIM = 64
BATCH = 1024
N_BLK = 2048


@functools.lru_cache(maxsize=None)
def _make_gather():
    info = plsc.get_sparse_core_info()
    nc, ns = info.num_cores, info.num_subcores
    nw = nc * ns
    b_per_w = BATCH // nw
    p_per_w = b_per_w // 2
    mesh = plsc.VectorSubcoreMesh(core_axis_name="c", subcore_axis_name="s")

    @functools.partial(
        pl.kernel,
        mesh=mesh,
        out_type=jax.ShapeDtypeStruct((BATCH // 2, 2 * EMB_DIM), jnp.float32),
        scratch_types=[
            pltpu.VMEM((b_per_w,), jnp.int32),
            pltpu.VMEM((b_per_w, 2 * EMB_DIM), jnp.float32),
            pltpu.VMEM((p_per_w, 2 * EMB_DIM), jnp.float32),
            pltpu.SemaphoreType.DMA,
        ],
        compiler_params=pltpu.CompilerParams(
            use_tc_tiling_on_sc=True, needs_layout_passes=False),
    )
    def gather_kernel(tbl_hbm, idx_hbm, out_hbm, idx_v, rows_v,
                      sel_v, sem):
        wid = lax.axis_index("s") * nc + lax.axis_index("c")
        base = wid * b_per_w
        pltpu.sync_copy(idx_hbm.at[pl.ds(base, b_per_w)], idx_v)
        pltpu.async_copy(tbl_hbm.at[idx_v], rows_v, sem).wait()
        iota = lax.iota(jnp.int32, 16)
        for j0 in range(0, b_per_w, 16):
            jv = iota + j0
            dst_row = lax.shift_right_logical(jv, 1)
            dst_col0 = (jv & 1) * EMB_DIM
            for c in range(EMB_DIM):
                val = plsc.load_gather(rows_v, [jv, iota * 0 + c])
                plsc.store_scatter(sel_v, [dst_row, dst_col0 + c], val)
        pltpu.sync_copy(sel_v, out_hbm.at[pl.ds(wid * p_per_w, p_per_w)])

    return gather_kernel


def _gather(indices, embedding_weight):
    tbl = jnp.concatenate(
        [embedding_weight,
         jnp.zeros((VOCAB, EMB_DIM), jnp.float32)], axis=1)
    packed = _make_gather()(tbl, indices)
    return packed.reshape(BATCH, EMB_DIM)


def _matmul_block(wt_ref, emb_ref, out_ref):
    wt = wt_ref[...].astype(jnp.bfloat16)
    e = emb_ref[...].astype(jnp.bfloat16)
    out_ref[...] = lax.dot_general(
        wt, e, (((0,), (1,)), ((), ())), preferred_element_type=jnp.float32)


def _project(emb, emb2vocab_weight, interpret=False):
    out_t = pl.pallas_call(
        _matmul_block,
        grid=(pl.cdiv(VOCAB, N_BLK),),
        in_specs=[
            pl.BlockSpec((EMB_DIM, N_BLK), lambda i: (0, i)),
            pl.BlockSpec((BATCH, EMB_DIM), lambda i: (0, 0)),
        ],
        out_specs=pl.BlockSpec((N_BLK, BATCH), lambda i: (i, 0)),
        out_shape=jax.ShapeDtypeStruct((VOCAB, BATCH), jnp.float32),
        compiler_params=pltpu.CompilerParams(
            vmem_limit_bytes=100 * 1024 * 1024,
        ),
        interpret=interpret,
    )(emb2vocab_weight.T, emb)
    return out_t.T


def kernel(indices, embedding_weight, emb2vocab_weight):
    emb = _gather(indices, embedding_weight)
    return _project(emb, emb2vocab_weight)


# speedup vs baseline: 4.9303x; 1.0022x over previous
import functools

import jax
import jax.numpy as jnp
from jax import lax
from jax.experimental import pallas as pl
from jax.experimental.pallas import tpu as pltpu
from jax.experimental.pallas import tpu_sc as plsc

VOCAB = 100000
EMB_DIM = 64
BATCH = 1024
N_BLK = 2048


@functools.lru_cache(maxsize=None)
def _make_gather():
    info = plsc.get_sparse_core_info()
    nc, ns = info.num_cores, info.num_subcores
    nw = nc * ns
    b_per_w = BATCH // nw
    p_per_w = b_per_w // 2
    mesh = plsc.VectorSubcoreMesh(core_axis_name="c", subcore_axis_name="s")

    @functools.partial(
        pl.kernel,
        mesh=mesh,
        out_type=jax.ShapeDtypeStruct((BATCH // 2, 2 * EMB_DIM), jnp.float32),
        scratch_types=[
            pltpu.VMEM((b_per_w,), jnp.int32),
            pltpu.VMEM((b_per_w, 2 * EMB_DIM), jnp.float32),
            pltpu.VMEM((p_per_w, 2 * EMB_DIM), jnp.float32),
            pltpu.SemaphoreType.DMA,
        ],
        compiler_params=pltpu.CompilerParams(
            use_tc_tiling_on_sc=False, needs_layout_passes=False),
    )
    def gather_kernel(tbl_hbm, idx_hbm, out_hbm, idx_v, rows_v,
                      sel_v, sem):
        wid = lax.axis_index("s") * nc + lax.axis_index("c")
        base = wid * b_per_w
        pltpu.sync_copy(idx_hbm.at[pl.ds(base, b_per_w)], idx_v)
        pltpu.async_copy(tbl_hbm.at[idx_v], rows_v, sem).wait()
        iota = lax.iota(jnp.int32, 16)
        for j0 in range(0, b_per_w, 16):
            jv = iota + j0
            dst_row = lax.shift_right_logical(jv, 1)
            dst_col0 = (jv & 1) * EMB_DIM
            for c in range(EMB_DIM):
                val = plsc.load_gather(rows_v, [jv, iota * 0 + c])
                plsc.store_scatter(sel_v, [dst_row, dst_col0 + c], val)
        pltpu.sync_copy(sel_v, out_hbm.at[pl.ds(wid * p_per_w, p_per_w)])

    return gather_kernel


def _gather(indices, embedding_weight):
    tbl = jnp.concatenate(
        [embedding_weight,
         jnp.zeros((VOCAB, EMB_DIM), jnp.float32)], axis=1)
    packed = _make_gather()(tbl, indices)
    return packed.reshape(BATCH, EMB_DIM)


def _matmul_block(wt_ref, emb_ref, out_ref):
    wt = wt_ref[...].astype(jnp.bfloat16)
    e = emb_ref[...].astype(jnp.bfloat16)
    out_ref[...] = lax.dot_general(
        wt, e, (((0,), (1,)), ((), ())), preferred_element_type=jnp.float32)


def _project(emb, emb2vocab_weight, interpret=False):
    out_t = pl.pallas_call(
        _matmul_block,
        grid=(pl.cdiv(VOCAB, N_BLK),),
        in_specs=[
            pl.BlockSpec((EMB_DIM, N_BLK), lambda i: (0, i)),
            pl.BlockSpec((BATCH, EMB_DIM), lambda i: (0, 0)),
        ],
        out_specs=pl.BlockSpec((N_BLK, BATCH), lambda i: (i, 0)),
        out_shape=jax.ShapeDtypeStruct((VOCAB, BATCH), jnp.float32),
        compiler_params=pltpu.CompilerParams(
            vmem_limit_bytes=100 * 1024 * 1024,
        ),
        interpret=interpret,
    )(emb2vocab_weight.T, emb)
    return out_t.T


def kernel(indices, embedding_weight, emb2vocab_weight):
    emb = _gather(indices, embedding_weight)
    return _project(emb, emb2vocab_weight)
